# baseline (speedup 1.0000x reference)
import functools

import jax
import jax.numpy as jnp
from jax import lax
from jax.experimental import pallas as pl
from jax.experimental.pallas import tpu as pltpu
from jax.experimental.pallas import tpu_sc as plsc

_SEQ = 1
_B = 10
_MAXN = 1000
_MAXE = 16000
_NF = 7
_EMB = 64
_NL = 5
_FLAT = _NF * _MAXN + 2 * _MAXE + _MAXN + 5

_NP = 1024
_NG = _SEQ * _B
_NPAD = _NG * _NP
_GSC = _NG // 2
_TPG = 3
_EPT = 6144
_CH = 128
_NCH = _EPT // _CH
_NEG = -1e30

_GDN = lax.GatherDimensionNumbers(
    offset_dims=(), collapsed_slice_dims=(0,), start_index_map=(0,))


def _hsum(v, iota):
    for sh in (8, 4, 2, 1):
        idx = (iota ^ sh)[:, None]
        v = v + lax.gather(v, idx, _GDN, (1,),
                           mode=lax.GatherScatterMode.PROMISE_IN_BOUNDS)
    return v


def _mm_first_kernel(x_ref, w_ref, b_ref, o_ref):
    o_ref[...] = x_ref[...] @ w_ref[...] + b_ref[...]


def _mm_mid_kernel(p_ref, bprev_ref, w_ref, b_ref, o_ref):
    x = p_ref[0] + p_ref[1] + p_ref[2] + bprev_ref[...]
    x = jnp.maximum(x, 0.0)
    o_ref[...] = x @ w_ref[...] + b_ref[...]


def _final_kernel(p_ref, bprev_ref, o_ref):
    o_ref[...] = p_ref[0] + p_ref[1] + p_ref[2] + bprev_ref[...]


def _wcat(p):
    return (jnp.concatenate([p['Wl'], p['Wr']], axis=1),
            jnp.concatenate([p['bl'], p['br']])[None, :])


def _mm_first(x, p):
    w, b = _wcat(p)
    return pl.pallas_call(
        _mm_first_kernel,
        out_shape=jax.ShapeDtypeStruct((_NPAD, 2 * _EMB), jnp.float32),
    )(x, w, b)


def _mm_mid(parts, bias_prev, p):
    w, b = _wcat(p)
    return pl.pallas_call(
        _mm_mid_kernel,
        out_shape=jax.ShapeDtypeStruct((_NPAD, 2 * _EMB), jnp.float32),
    )(parts, bias_prev[None, :], w, b)


def _final_sum(parts, bias_prev):
    return pl.pallas_call(
        _final_kernel,
        out_shape=jax.ShapeDtypeStruct((_NPAD, _EMB), jnp.float32),
    )(parts, bias_prev[None, :])


def _edge_kernel(xlr_hbm, iarr_hbm, att_hbm,
                 out_hbm,
                 iarr_v, buf_s, buf_d, alpha_v, tabs, amax_v, den_v,
                 att_v, idxs_b, idxd_b, out_loc,
                 stage_sh, red_sh, sem0, sem1):
    ci = lax.axis_index("c")
    si = lax.axis_index("s")
    g_local = jnp.minimum(si // _TPG, _GSC - 1)
    r = si - g_local * _TPG
    gbase = (ci * _GSC + g_local) * _NP
    iota = lax.iota(jnp.int32, 16)

    pltpu.sync_copy(iarr_hbm.at[ci, si], iarr_v)
    pltpu.sync_copy(att_hbm, att_v)

    def init_tab(val):
        def body(i, _):
            tabs[pl.ds(i * 16, 16)] = jnp.full((16,), val, jnp.float32)
            return 0
        lax.fori_loop(0, (16 * _NP) // 16, body, 0)

    init_tab(_NEG)

    def stage_idx(j):
        for q in range(8):
            v = iarr_v[j, pl.ds(q * 16, 16)]
            ds = pl.ds(q * 16, 16)
            idxs_b[ds] = v & 16383
            idxd_b[ds] = v >> 14

    def alpha_chunk(j, _):
        if True:
            hb = 0
            stage_idx(j)
            cp0 = pltpu.async_copy(xlr_hbm.at[idxs_b], buf_s, sem0)
            cp1 = pltpu.async_copy(xlr_hbm.at[idxd_b], buf_d, sem1)
            cp0.wait()
            cp1.wait()

            def grp_body(g, _):
                dv16 = idxd_b[pl.ds(g * 16, 16)] - gbase
                alphav = jnp.zeros((16,), jnp.float32)
                for lane in range(16):
                    e = g * 16 + lane
                    acc = jnp.zeros((16,), jnp.float32)
                    for k in range(4):
                        dk = pl.ds(k * 16, 16)
                        s = buf_s[e, dk] + buf_d[e, pl.ds(_EMB + k * 16, 16)]
                        lr = jnp.maximum(s, s * 0.2)
                        acc = acc + att_v[dk] * lr
                    s16 = _hsum(acc, iota)
                    alphav = jnp.where(iota == lane, s16, alphav)
                alpha_v[pl.ds(j * 128 + hb + g * 16, 16)] = alphav
                tabidx = iota * _NP + dv16
                m = plsc.load_gather(tabs, [tabidx])
                plsc.store_scatter(tabs, [tabidx], jnp.maximum(m, alphav))
                return 0

            lax.fori_loop(0, 8, grp_body, 0, unroll=2)
        return 0

    lax.fori_loop(0, _NCH, alpha_chunk, 0)

    def fold_combine(res_v, scr_v, combine_fn):
        def fold(i, _):
            v = tabs[pl.ds(i * 16, 16)]
            for l in range(1, 16):
                v = combine_fn(v, tabs[pl.ds(l * _NP + i * 16, 16)])
            res_v[pl.ds(i * 16, 16)] = v
            return 0
        lax.fori_loop(0, _NP // 16, fold, 0)
        pltpu.sync_copy(res_v, stage_sh.at[si])
        plsc.subcore_barrier()

        @pl.when(r == 0)
        def _():
            for t in (1, 2):
                pltpu.sync_copy(stage_sh.at[si + t], scr_v)

                def merge(i, _):
                    ds = pl.ds(i * 16, 16)
                    res_v[ds] = combine_fn(res_v[ds], scr_v[ds])
                    return 0
                lax.fori_loop(0, _NP // 16, merge, 0)
            pltpu.sync_copy(res_v, red_sh.at[g_local])
        plsc.subcore_barrier()
        pltpu.sync_copy(red_sh.at[g_local], res_v)

    fold_combine(amax_v, den_v, jnp.maximum)

    init_tab(0.0)

    def exp_chunk(j, _):
        if True:
            hb = 0
            stage_idx(j)

            def grp_body(g, _):
                dsa = pl.ds(j * 128 + hb + g * 16, 16)
                al = alpha_v[dsa]
                dv16 = idxd_b[pl.ds(g * 16, 16)] - gbase
                am = plsc.load_gather(amax_v, [dv16])
                p = jnp.exp(al - am)
                alpha_v[dsa] = p
                tabidx = iota * _NP + dv16
                sv = plsc.load_gather(tabs, [tabidx])
                plsc.store_scatter(tabs, [tabidx], sv + p)
                return 0
            lax.fori_loop(0, 8, grp_body, 0)
        return 0

    lax.fori_loop(0, _NCH, exp_chunk, 0)

    fold_combine(den_v, amax_v, lambda a, b: a + b)

    def zero_out(n, _):
        for k in range(4):
            out_loc[pl.ds(n * _EMB + k * 16, 16)] = jnp.zeros((16,),
                                                              jnp.float32)
        return 0
    lax.fori_loop(0, _NP, zero_out, 0)

    def accum_chunk(j, _):
        if True:
            hb = 0
            stage_idx(j)
            pltpu.async_copy(xlr_hbm.at[idxs_b], buf_s, sem0).wait()

            def grp_body(g, _):
                p16 = alpha_v[pl.ds(j * 128 + hb + g * 16, 16)]
                dv16 = idxd_b[pl.ds(g * 16, 16)] - gbase
                dn16 = plsc.load_gather(den_v, [dv16])
                w16 = p16 / dn16
                for lane in range(16):
                    d = dv16[lane]
                    w = w16[lane]
                    e = g * 16 + lane
                    for k in range(4):
                        dsk = pl.ds(d * _EMB + k * 16, 16)
                        out_loc[dsk] = out_loc[dsk] \
                            + w * buf_s[e, pl.ds(k * 16, 16)]
                return 0

            lax.fori_loop(0, 8, grp_body, 0, unroll=2)
        return 0

    lax.fori_loop(0, _NCH, accum_chunk, 0)

    @pl.when(r < _TPG)
    def _():
        for q in range(4):
            def pack(n, _):
                b = (q * 256 + n * 2) * _EMB
                for k in range(4):
                    buf_s[n, pl.ds(k * 16, 16)] = \
                        out_loc[pl.ds(b + k * 16, 16)]
                    buf_s[n, pl.ds(_EMB + k * 16, 16)] = \
                        out_loc[pl.ds(b + _EMB + k * 16, 16)]
                return 0
            lax.fori_loop(0, 128, pack, 0)
            pltpu.sync_copy(
                buf_s,
                out_hbm.at[pl.ds(r * (_GSC * _NP)
                                 + (ci * _GSC + g_local) * (_NP // 2)
                                 + q * 128, 128)])


def _edge_layer(xlr, iarr, att128):
    mesh = plsc.VectorSubcoreMesh(core_axis_name="c", subcore_axis_name="s")
    f = pl.kernel(
        _edge_kernel,
        mesh=mesh,
        compiler_params=pltpu.CompilerParams(needs_layout_passes=False),
        out_type=jax.ShapeDtypeStruct((_TPG * _GSC * _NP, 2 * _EMB),
                                      jnp.float32),
        scratch_types=[
            pltpu.VMEM((_EPT // 128, 128), jnp.int32),
            pltpu.VMEM((128, 2 * _EMB), jnp.float32),
            pltpu.VMEM((128, 2 * _EMB), jnp.float32),
            pltpu.VMEM((_EPT,), jnp.float32),
            pltpu.VMEM((16 * _NP,), jnp.float32),
            pltpu.VMEM((_NP,), jnp.float32),
            pltpu.VMEM((_NP,), jnp.float32),
            pltpu.VMEM((2 * _EMB,), jnp.float32),
            pltpu.VMEM((128,), jnp.int32),
            pltpu.VMEM((128,), jnp.int32),
            pltpu.VMEM((_NP * _EMB,), jnp.float32),
            pltpu.VMEM_SHARED((16, _NP), jnp.float32),
            pltpu.VMEM_SHARED((_GSC, _NP), jnp.float32),
            pltpu.SemaphoreType.DMA,
            pltpu.SemaphoreType.DMA,
        ],
    )
    return f(xlr, iarr, att128)


def _build_edge_indices(py):
    loops = jnp.broadcast_to(jnp.arange(_MAXN, dtype=jnp.int32), (_NG, _MAXN))
    pad = jnp.full((_NG, _TPG * _EPT - _MAXE - _MAXN), _MAXN, jnp.int32)
    src = jnp.concatenate([py[:, 0, :], loops, pad], axis=1)
    dst = jnp.concatenate([py[:, 1, :], loops, pad], axis=1)

    def arrange(a):
        a = a + (jnp.arange(_NG, dtype=jnp.int32) * _NP)[:, None]
        a = a.reshape(2, _GSC * _TPG, _EPT)
        filler = jnp.broadcast_to(
            (jnp.arange(2, dtype=jnp.int32) * (_GSC * _NP)
             + (_GSC - 1) * _NP + _MAXN)[:, None, None],
            (2, 1, _EPT)).astype(jnp.int32)
        a = jnp.concatenate([a, filler], axis=1)
        return a.reshape(2, 16, _NCH, 128)

    return arrange(src) + arrange(dst) * 16384


def kernel(state, params):
    flat = state.reshape(-1, _FLAT)
    nf = flat[:, :_NF * _MAXN].reshape(_NG, _MAXN, _NF)
    py = flat[:, _NF * _MAXN:_NF * _MAXN + 2 * _MAXE].reshape(_NG, 2, _MAXE)
    py = py.astype(jnp.int32)
    reach = flat[:, _NF * _MAXN + 2 * _MAXE:_NF * _MAXN + 2 * _MAXE + _MAXN]
    reach = reach.reshape(-1)

    iarr = _build_edge_indices(py)

    hp = jnp.pad(nf, ((0, 0), (0, _NP - _MAXN), (0, 0))).reshape(_NPAD, _NF)

    xlr = _mm_first(hp, params[0])
    h2 = None
    for l in range(_NL):
        att128 = jnp.pad(params[l]['att'], (0, _EMB))
        part = _edge_layer(xlr, iarr, att128)
        parts = part.reshape(_TPG, _NPAD, _EMB)
        if l < _NL - 1:
            xlr = _mm_mid(parts, params[l]['bias'], params[l + 1])
        else:
            h2 = _final_sum(parts, params[l]['bias'])

    h = h2.reshape(_NG, _NP, _EMB)[:, :_MAXN].reshape(_NG * _MAXN, _EMB)

    N = _NG * _MAXN
    batch_vec = jnp.repeat(jnp.arange(_NG), _MAXN).astype(jnp.float32)
    num_nodes_vec = jnp.concatenate([
        jnp.full((_NG,), float(_MAXN), dtype=jnp.float32),
        jnp.zeros((N - _NG,), jnp.float32),
    ])
    features = jnp.concatenate(
        [h, batch_vec[:, None], reach[:, None], num_nodes_vec[:, None]],
        axis=1)
    features = features.reshape(_SEQ, N, _EMB + 3)
    g = jnp.arange(_NG, dtype=jnp.int64)
    valid_entries_idx = jnp.stack([g * _MAXN, g * _MAXN + _MAXN], axis=1)
    return (features, jnp.array(N), valid_entries_idx, num_nodes_vec)


# speedup vs baseline: 1.4268x; 1.4268x over previous
import jax
import jax.numpy as jnp
from jax import lax
from jax.experimental import pallas as pl
from jax.experimental.pallas import tpu as pltpu
from jax.experimental.pallas import tpu_sc as plsc

_SEQ = 1
_B = 10
_MAXN = 1000
_MAXE = 16000
_NF = 7
_EMB = 64
_NL = 5
_FLAT = _NF * _MAXN + 2 * _MAXE + _MAXN + 5

_NP = 1024
_NG = _SEQ * _B
_NPAD = _NG * _NP
_GSC = _NG // 2
_TPG = 3
_EPT = 6144
_NCH = _EPT // 128

_GDN = lax.GatherDimensionNumbers(
    offset_dims=(), collapsed_slice_dims=(0,), start_index_map=(0,))


def _hsum(v, iota):
    for sh in (8, 4, 2, 1):
        idx = (iota ^ sh)[:, None]
        v = v + lax.gather(v, idx, _GDN, (1,),
                           mode=lax.GatherScatterMode.PROMISE_IN_BOUNDS)
    return v


def _stab(xlr, att):
    s = xlr[:, :_EMB] + xlr[:, _EMB:]
    return jnp.sum(jnp.maximum(s, s * 0.2) * att, axis=1)[None, :]


def _mm_first_kernel(x_ref, w_ref, b_ref, att_ref, o_ref, c_ref):
    xlr = x_ref[...] @ w_ref[...] + b_ref[...]
    o_ref[...] = xlr
    c_ref[...] = _stab(xlr, att_ref[...])


def _mm_mid_kernel(p_ref, d_ref, bprev_ref, w_ref, b_ref, att_ref,
                   o_ref, c_ref):
    den = d_ref[0] + d_ref[1] + d_ref[2] + 1e-30
    x = (p_ref[0] + p_ref[1] + p_ref[2]) / den[:, None] + bprev_ref[...]
    x = jnp.maximum(x, 0.0)
    xlr = x @ w_ref[...] + b_ref[...]
    o_ref[...] = xlr
    c_ref[...] = _stab(xlr, att_ref[...])


def _final_kernel(p_ref, d_ref, bprev_ref, o_ref):
    den = d_ref[0] + d_ref[1] + d_ref[2] + 1e-30
    o_ref[...] = ((p_ref[0] + p_ref[1] + p_ref[2]) / den[:, None]
                  + bprev_ref[...])


def _wcat(p):
    return (jnp.concatenate([p['Wl'], p['Wr']], axis=1),
            jnp.concatenate([p['bl'], p['br']])[None, :])


def _mm_first(x, p):
    w, b = _wcat(p)
    return pl.pallas_call(
        _mm_first_kernel,
        out_shape=(jax.ShapeDtypeStruct((_NPAD, 2 * _EMB), jnp.float32),
                   jax.ShapeDtypeStruct((1, _NPAD), jnp.float32)),
    )(x, w, b, p['att'][None, :])


def _mm_mid(parts, dens, bias_prev, p):
    w, b = _wcat(p)
    return pl.pallas_call(
        _mm_mid_kernel,
        out_shape=(jax.ShapeDtypeStruct((_NPAD, 2 * _EMB), jnp.float32),
                   jax.ShapeDtypeStruct((1, _NPAD), jnp.float32)),
    )(parts, dens, bias_prev[None, :], w, b, p['att'][None, :])


def _final_sum(parts, dens, bias_prev):
    return pl.pallas_call(
        _final_kernel,
        out_shape=jax.ShapeDtypeStruct((_NPAD, _EMB), jnp.float32),
    )(parts, dens, bias_prev[None, :])


def _edge_kernel(xlr_hbm, iarr_hbm, att_hbm, c_hbm,
                 out_hbm, den_hbm,
                 iarr_v, buf_s, buf_d, tabs, c_v, den_v, att_v,
                 idxs_b, idxd_b, out_loc, sem0, sem1):
    ci = lax.axis_index("c")
    si = lax.axis_index("s")
    g_local = jnp.minimum(si // _TPG, _GSC - 1)
    r = si - g_local * _TPG
    gbase = (ci * _GSC + g_local) * _NP
    iota = lax.iota(jnp.int32, 16)

    pltpu.sync_copy(iarr_hbm.at[ci, si], iarr_v)
    pltpu.sync_copy(att_hbm, att_v)
    pltpu.sync_copy(c_hbm.at[0, pl.ds(gbase, _NP)], c_v)

    def init_tab(i, _):
        tabs[pl.ds(i * 16, 16)] = jnp.zeros((16,), jnp.float32)
        return 0
    lax.fori_loop(0, (16 * _NP) // 16, init_tab, 0)

    def zero_out(n, _):
        for k in range(4):
            out_loc[pl.ds(n * _EMB + k * 16, 16)] = jnp.zeros((16,),
                                                              jnp.float32)
        return 0
    lax.fori_loop(0, _NP, zero_out, 0)

    def stage_idx(j):
        for q in range(8):
            v = iarr_v[j, pl.ds(q * 16, 16)]
            ds = pl.ds(q * 16, 16)
            idxs_b[ds] = v & 16383
            idxd_b[ds] = v >> 14

    def chunk(j, _):
        stage_idx(j)
        cp0 = pltpu.async_copy(xlr_hbm.at[idxs_b], buf_s, sem0)
        cp1 = pltpu.async_copy(xlr_hbm.at[idxd_b], buf_d, sem1)
        cp0.wait()
        cp1.wait()

        def grp_body(g, _):
            dv16 = idxd_b[pl.ds(g * 16, 16)] - gbase
            alphav = jnp.zeros((16,), jnp.float32)
            for lane in range(16):
                e = g * 16 + lane
                acc = jnp.zeros((16,), jnp.float32)
                for k in range(4):
                    dk = pl.ds(k * 16, 16)
                    s = buf_s[e, dk] + buf_d[e, pl.ds(_EMB + k * 16, 16)]
                    lr = jnp.maximum(s, s * 0.2)
                    acc = acc + att_v[dk] * lr
                s16 = _hsum(acc, iota)
                alphav = jnp.where(iota == lane, s16, alphav)
            cv = plsc.load_gather(c_v, [dv16])
            p16 = jnp.exp(alphav - cv)
            tabidx = iota * _NP + dv16
            sv = plsc.load_gather(tabs, [tabidx])
            plsc.store_scatter(tabs, [tabidx], sv + p16)
            for lane in range(16):
                d = dv16[lane]
                w = p16[lane]
                e = g * 16 + lane
                for k in range(4):
                    dsk = pl.ds(d * _EMB + k * 16, 16)
                    out_loc[dsk] = out_loc[dsk] \
                        + w * buf_s[e, pl.ds(k * 16, 16)]
            return 0

        lax.fori_loop(0, 8, grp_body, 0)
        return 0

    lax.fori_loop(0, _NCH, chunk, 0)

    def fold(i, _):
        v = tabs[pl.ds(i * 16, 16)]
        for l in range(1, 16):
            v = v + tabs[pl.ds(l * _NP + i * 16, 16)]
        den_v[pl.ds(i * 16, 16)] = v
        return 0
    lax.fori_loop(0, _NP // 16, fold, 0)

    @pl.when(r < _TPG)
    def _():
        pltpu.sync_copy(den_v,
                        den_hbm.at[0, pl.ds(r * _NPAD + gbase, _NP)])
        for q in range(4):
            def pack(n, _):
                b = (q * 256 + n * 2) * _EMB
                for k in range(4):
                    buf_s[n, pl.ds(k * 16, 16)] = \
                        out_loc[pl.ds(b + k * 16, 16)]
                    buf_s[n, pl.ds(_EMB + k * 16, 16)] = \
                        out_loc[pl.ds(b + _EMB + k * 16, 16)]
                return 0
            lax.fori_loop(0, 128, pack, 0)
            pltpu.sync_copy(
                buf_s,
                out_hbm.at[pl.ds(r * (_GSC * _NP)
                                 + (ci * _GSC + g_local) * (_NP // 2)
                                 + q * 128, 128)])


def _edge_layer(xlr, iarr, att128, c):
    mesh = plsc.VectorSubcoreMesh(core_axis_name="c", subcore_axis_name="s")
    f = pl.kernel(
        _edge_kernel,
        mesh=mesh,
        compiler_params=pltpu.CompilerParams(needs_layout_passes=False),
        out_type=(jax.ShapeDtypeStruct((_TPG * _GSC * _NP, 2 * _EMB),
                                       jnp.float32),
                  jax.ShapeDtypeStruct((1, _TPG * _NPAD), jnp.float32)),
        scratch_types=[
            pltpu.VMEM((_NCH, 128), jnp.int32),
            pltpu.VMEM((128, 2 * _EMB), jnp.float32),
            pltpu.VMEM((128, 2 * _EMB), jnp.float32),
            pltpu.VMEM((16 * _NP,), jnp.float32),
            pltpu.VMEM((_NP,), jnp.float32),
            pltpu.VMEM((_NP,), jnp.float32),
            pltpu.VMEM((2 * _EMB,), jnp.float32),
            pltpu.VMEM((128,), jnp.int32),
            pltpu.VMEM((128,), jnp.int32),
            pltpu.VMEM((_NP * _EMB,), jnp.float32),
            pltpu.SemaphoreType.DMA,
            pltpu.SemaphoreType.DMA,
        ],
    )
    return f(xlr, iarr, att128, c)


def _build_edge_indices(py):
    loops = jnp.broadcast_to(jnp.arange(_MAXN, dtype=jnp.int32), (_NG, _MAXN))
    pad = jnp.full((_NG, _TPG * _EPT - _MAXE - _MAXN), _MAXN, jnp.int32)
    src = jnp.concatenate([py[:, 0, :], loops, pad], axis=1)
    dst = jnp.concatenate([py[:, 1, :], loops, pad], axis=1)

    def arrange(a):
        a = a + (jnp.arange(_NG, dtype=jnp.int32) * _NP)[:, None]
        a = a.reshape(2, _GSC * _TPG, _EPT)
        filler = jnp.broadcast_to(
            (jnp.arange(2, dtype=jnp.int32) * (_GSC * _NP)
             + (_GSC - 1) * _NP + _MAXN)[:, None, None],
            (2, 1, _EPT)).astype(jnp.int32)
        a = jnp.concatenate([a, filler], axis=1)
        return a.reshape(2, 16, _NCH, 128)

    return arrange(src) + arrange(dst) * 16384


def kernel(state, params):
    flat = state.reshape(-1, _FLAT)
    nf = flat[:, :_NF * _MAXN].reshape(_NG, _MAXN, _NF)
    py = flat[:, _NF * _MAXN:_NF * _MAXN + 2 * _MAXE].reshape(_NG, 2, _MAXE)
    py = py.astype(jnp.int32)
    reach = flat[:, _NF * _MAXN + 2 * _MAXE:_NF * _MAXN + 2 * _MAXE + _MAXN]
    reach = reach.reshape(-1)

    iarr = _build_edge_indices(py)

    hp = jnp.pad(nf, ((0, 0), (0, _NP - _MAXN), (0, 0))).reshape(_NPAD, _NF)

    xlr, c = _mm_first(hp, params[0])
    h2 = None
    for l in range(_NL):
        att128 = jnp.pad(params[l]['att'], (0, _EMB))
        part, den = _edge_layer(xlr, iarr, att128, c)
        parts = part.reshape(_TPG, _NPAD, _EMB)
        dens = den.reshape(_TPG, _NPAD)
        if l < _NL - 1:
            xlr, c = _mm_mid(parts, dens, params[l]['bias'], params[l + 1])
        else:
            h2 = _final_sum(parts, dens, params[l]['bias'])

    h = h2.reshape(_NG, _NP, _EMB)[:, :_MAXN].reshape(_NG * _MAXN, _EMB)

    N = _NG * _MAXN
    batch_vec = jnp.repeat(jnp.arange(_NG), _MAXN).astype(jnp.float32)
    num_nodes_vec = jnp.concatenate([
        jnp.full((_NG,), float(_MAXN), dtype=jnp.float32),
        jnp.zeros((N - _NG,), jnp.float32),
    ])
    features = jnp.concatenate(
        [h, batch_vec[:, None], reach[:, None], num_nodes_vec[:, None]],
        axis=1)
    features = features.reshape(_SEQ, N, _EMB + 3)
    g = jnp.arange(_NG, dtype=jnp.int64)
    valid_entries_idx = jnp.stack([g * _MAXN, g * _MAXN + _MAXN], axis=1)
    return (features, jnp.array(N), valid_entries_idx, num_nodes_vec)
